# baseline (speedup 1.0000x reference)
import jax
import jax.numpy as jnp
from jax.experimental import pallas as pl
from jax.experimental.pallas import tpu as pltpu

_NCHUNK = 8


def _paged_update(slots, keep, tok_k, tok_v, kc, vc):
    num_slots, row = kc.shape
    n_tok = tok_k.shape[0]
    chunk = num_slots // _NCHUNK

    def body(slots_ref, keep_ref, kin, vin, tk, tv, kout, vout,
             sem_bulk, sem_row):
        def bulk(src, dst, j):
            rows = pl.ds((j % _NCHUNK) * chunk, chunk)
            return pltpu.make_async_copy(
                src.at[rows], dst.at[rows], sem_bulk.at[j])

        for c in range(_NCHUNK):
            bulk(kin, kout, c).start()
            bulk(vin, vout, c + _NCHUNK).start()
        for c in range(_NCHUNK):
            bulk(kin, kout, c).wait()
            bulk(vin, vout, c + _NCHUNK).wait()

        def rowcp(t):
            s = slots_ref[t]
            return (
                pltpu.make_async_copy(tk.at[t], kout.at[s], sem_row.at[0, t]),
                pltpu.make_async_copy(tv.at[t], vout.at[s], sem_row.at[1, t]),
            )

        for t in range(n_tok):
            @pl.when(keep_ref[t] != 0)
            def _():
                ck, cv = rowcp(t)
                ck.start()
                cv.start()
        for t in range(n_tok):
            @pl.when(keep_ref[t] != 0)
            def _():
                ck, cv = rowcp(t)
                ck.wait()
                cv.wait()

    anyspec = pl.BlockSpec(memory_space=pl.ANY)
    return pl.pallas_call(
        body,
        grid_spec=pltpu.PrefetchScalarGridSpec(
            num_scalar_prefetch=2,
            grid=(1,),
            in_specs=[anyspec, anyspec, anyspec, anyspec],
            out_specs=[anyspec, anyspec],
            scratch_shapes=[
                pltpu.SemaphoreType.DMA((2 * _NCHUNK,)),
                pltpu.SemaphoreType.DMA((2, n_tok)),
            ],
        ),
        out_shape=(
            jax.ShapeDtypeStruct(kc.shape, kc.dtype),
            jax.ShapeDtypeStruct(vc.shape, vc.dtype),
        ),
        compiler_params=pltpu.CompilerParams(
            dimension_semantics=("arbitrary",),
        ),
    )(slots, keep, kc, vc, tok_k, tok_v)


def kernel(pos_ids, k_val, v_val, slot_mapping, batch_idx, k_cache, v_cache):
    B, H, S, D = k_val.shape
    tok_k = jnp.transpose(k_val, (0, 2, 1, 3)).reshape(B * S, H * D)
    tok_v = jnp.transpose(v_val, (0, 2, 1, 3)).reshape(B * S, H * D)
    kc = k_cache.reshape(k_cache.shape[0], H * D)
    vc = v_cache.reshape(v_cache.shape[0], H * D)
    n = slot_mapping.shape[0]
    later = (slot_mapping[None, :] == slot_mapping[:, None]) & (
        jnp.arange(n)[None, :] > jnp.arange(n)[:, None])
    keep = (~jnp.any(later, axis=1)).astype(jnp.int32)
    ko, vo = _paged_update(slot_mapping, keep, tok_k, tok_v, kc, vc)
    return ko.reshape(k_cache.shape), vo.reshape(v_cache.shape)
